# baseline (speedup 1.0000x reference)
import jax
import jax.numpy as jnp
from jax import lax
from jax.experimental import pallas as pl
from jax.experimental.pallas import tpu as pltpu

_DIM = 1024
_E = 64
_T = 32768
_EPS = 1e-06
_BT = 2048


def _body(x_ref, wt_ref, b_ref, out_ref, colsum_ref):
    j = pl.program_id(0)
    s = jnp.dot(x_ref[...], wt_ref[...], preferred_element_type=jnp.float32)
    s = s + b_ref[0:1, :]
    m = jnp.max(s, axis=1, keepdims=True)
    e = jnp.exp(s - m)
    z = jnp.sum(e, axis=1, keepdims=True)
    v = 1.0 / z
    iota = lax.broadcasted_iota(jnp.int32, s.shape, 1)
    cand = jnp.where(s == m, iota, _E)
    idx = jnp.min(cand, axis=1, keepdims=True)
    masked = jnp.where(iota == idx, v, 0.0)
    out_ref[pl.ds(j * _BT, _BT), :] = masked
    part = jnp.sum(masked, axis=0, keepdims=True)

    @pl.when(j == 0)
    def _():
        colsum_ref[...] = jnp.zeros_like(colsum_ref)

    colsum_ref[...] += jnp.broadcast_to(part, colsum_ref.shape)

    @pl.when(j == pl.num_programs(0) - 1)
    def _():
        denom = colsum_ref[0:1, :] + _EPS
        out_ref[...] = out_ref[...] / denom * float(_T)


def kernel(x, W, b):
    wt = W.T
    b2 = jnp.broadcast_to(b.reshape(1, _E), (8, _E))

    out = pl.pallas_call(
        _body,
        grid=(_T // _BT,),
        in_specs=[
            pl.BlockSpec((_BT, _DIM), lambda i: (i, 0)),
            pl.BlockSpec((_DIM, _E), lambda i: (0, 0)),
            pl.BlockSpec((8, _E), lambda i: (0, 0)),
        ],
        out_specs=pl.BlockSpec((_T, _E), lambda i: (0, 0)),
        out_shape=jax.ShapeDtypeStruct((_T, _E), jnp.float32),
        scratch_shapes=[pltpu.VMEM((8, _E), jnp.float32)],
    )(x, wt, b2)
    return out


# speedup vs baseline: 1.0997x; 1.0997x over previous
import jax
import jax.numpy as jnp
from jax import lax
from jax.experimental import pallas as pl
from jax.experimental.pallas import tpu as pltpu

_DIM = 1024
_E = 64
_T = 32768
_EPS = 1e-06
_BT = 4096


def _body(x_ref, w_ref, b_ref, out_ref, colsum_ref, wt_ref):
    j = pl.program_id(0)

    @pl.when(j == 0)
    def _():
        wt_ref[...] = w_ref[...].T
        colsum_ref[...] = jnp.zeros_like(colsum_ref)

    s = jnp.dot(x_ref[...], wt_ref[...], preferred_element_type=jnp.float32)
    s = s + b_ref[0:1, :]
    m = jnp.max(s, axis=1, keepdims=True)
    e = jnp.exp(s - m)
    z = jnp.sum(e, axis=1, keepdims=True)
    v = 1.0 / z
    iota = lax.broadcasted_iota(jnp.int32, s.shape, 1).astype(jnp.float32)
    cand = jnp.where(s == m, iota, float(_E))
    idx = jnp.min(cand, axis=1, keepdims=True)
    masked = jnp.where(iota == idx, v, 0.0)
    out_ref[pl.ds(j * _BT, _BT), :] = masked
    part = jnp.sum(masked, axis=0, keepdims=True)
    colsum_ref[...] += jnp.broadcast_to(part, colsum_ref.shape)

    @pl.when(j == pl.num_programs(0) - 1)
    def _():
        denom = colsum_ref[0:1, :] + _EPS
        out_ref[...] = out_ref[...] / denom * float(_T)


def kernel(x, W, b):
    out = pl.pallas_call(
        _body,
        grid=(_T // _BT,),
        in_specs=[
            pl.BlockSpec((_BT, _DIM), lambda i: (i, 0)),
            pl.BlockSpec((_E, _DIM), lambda i: (0, 0)),
            pl.BlockSpec((1, _E), lambda i: (0, 0)),
        ],
        out_specs=pl.BlockSpec((_T, _E), lambda i: (0, 0)),
        out_shape=jax.ShapeDtypeStruct((_T, _E), jnp.float32),
        scratch_shapes=[
            pltpu.VMEM((8, _E), jnp.float32),
            pltpu.VMEM((_DIM, _E), jnp.float32),
        ],
    )(x, W, b.reshape(1, _E))
    return out


# speedup vs baseline: 1.1227x; 1.0209x over previous
import jax
import jax.numpy as jnp
from jax import lax
from jax.experimental import pallas as pl
from jax.experimental.pallas import tpu as pltpu

_DIM = 1024
_E = 64
_T = 32768
_EPS = 1e-06
_CR = 1024
_NB = 4
_NC = _T // _CR
_OR = 4096
_NO = _T // _OR


def _body(x_hbm, w_ref, b_ref, out_hbm, buf, insems, acc, stage, outsems,
          colsum_ref, wt_ref):
    def start_in(i, slot):
        pltpu.make_async_copy(
            x_hbm.at[pl.ds(i * _CR, _CR), :], buf.at[slot], insems.at[slot]
        ).start()

    def wait_in(i, slot):
        pltpu.make_async_copy(
            x_hbm.at[pl.ds(i * _CR, _CR), :], buf.at[slot], insems.at[slot]
        ).wait()

    for k in range(_NB):
        start_in(k, k)

    wt_ref[...] = w_ref[...].T
    colsum_ref[...] = jnp.zeros_like(colsum_ref)

    def step(i, carry):
        slot = lax.rem(i, _NB)
        wait_in(i, slot)
        s = jnp.dot(buf[slot], wt_ref[...],
                    preferred_element_type=jnp.float32)
        s = s + b_ref[0:1, :]
        m = jnp.max(s, axis=1, keepdims=True)
        e = jnp.exp(s - m)
        z = jnp.sum(e, axis=1, keepdims=True)
        v = 1.0 / z
        iota = lax.broadcasted_iota(jnp.int32, s.shape, 1).astype(jnp.float32)
        cand = jnp.where(s == m, iota, float(_E))
        idx = jnp.min(cand, axis=1, keepdims=True)
        masked = jnp.where(iota == idx, v, 0.0)
        acc[pl.ds(i * _CR, _CR), :] = masked
        part = jnp.sum(masked, axis=0, keepdims=True)
        colsum_ref[...] += jnp.broadcast_to(part, colsum_ref.shape)

        @pl.when(i + _NB < _NC)
        def _():
            start_in(i + _NB, slot)

        return carry

    lax.fori_loop(0, _NC, step, 0)

    scale = float(_T) / (colsum_ref[0:1, :] + _EPS)
    for c in range(_NO):
        slot = c % 2
        if c >= 2:
            pltpu.make_async_copy(
                stage.at[slot], out_hbm.at[pl.ds((c - 2) * _OR, _OR), :],
                outsems.at[slot],
            ).wait()
        stage[slot, :, :] = acc[pl.ds(c * _OR, _OR), :] * scale
        pltpu.make_async_copy(
            stage.at[slot], out_hbm.at[pl.ds(c * _OR, _OR), :],
            outsems.at[slot],
        ).start()
    for c in range(_NO - 2, _NO):
        slot = c % 2
        pltpu.make_async_copy(
            stage.at[slot], out_hbm.at[pl.ds(c * _OR, _OR), :],
            outsems.at[slot],
        ).wait()


def kernel(x, W, b):
    out = pl.pallas_call(
        _body,
        in_specs=[
            pl.BlockSpec(memory_space=pltpu.MemorySpace.HBM),
            pl.BlockSpec((_E, _DIM), lambda: (0, 0)),
            pl.BlockSpec((1, _E), lambda: (0, 0)),
        ],
        out_specs=pl.BlockSpec(memory_space=pltpu.MemorySpace.HBM),
        out_shape=jax.ShapeDtypeStruct((_T, _E), jnp.float32),
        scratch_shapes=[
            pltpu.VMEM((_NB, _CR, _DIM), jnp.float32),
            pltpu.SemaphoreType.DMA((_NB,)),
            pltpu.VMEM((_T, _E), jnp.float32),
            pltpu.VMEM((2, _OR, _E), jnp.float32),
            pltpu.SemaphoreType.DMA((2,)),
            pltpu.VMEM((8, _E), jnp.float32),
            pltpu.VMEM((_DIM, _E), jnp.float32),
        ],
    )(x, W, b.reshape(1, _E))
    return out
